# baseline (speedup 1.0000x reference)
import math

import numpy as np
import jax
import jax.numpy as jnp
from jax import lax
from jax.experimental import pallas as pl
from jax.experimental.pallas import tpu as pltpu, tpu_sc as plsc

_RC = 5.0
_RS = np.linspace(0.5, 4.5, 8).astype(np.float32)
_ETAS = np.array([0.5, 1.0, 2.0], dtype=np.float32)
_NFP = 24
_NATOMS = 10000
_NPAIRS = 320000

_NC, _NS = 2, 16
_NW = _NC * _NS
_ROWS_W = 10240
_P_PAD = _NW * _ROWS_W
_G = 1024
_NG = _ROWS_W // _G
_SUB = 128
_NSUB = _G // _SUB
_NATOMS_PAD = 10240
_AZ = _NATOMS_PAD // _NS


_SROWS = 640
_SCOLS = 512
_SB = 128
_FC_COEFFS = [1.0] + [0.5 * (-1.0) ** k / float(math.factorial(2 * k))
               for k in range(1, 9)]


def _scal_body(dxyz_ref, out_ref):
    dx = dxyz_ref[0]
    dy = dxyz_ref[1]
    dz = dxyz_ref[2]
    d2 = dx * dx + dy * dy + dz * dz
    dist = jnp.sqrt(d2 + 1e-12)
    u = d2 * np.float32((np.pi / _RC) ** 2)
    p = jnp.full_like(u, np.float32(_FC_COEFFS[-1]))
    for c in _FC_COEFFS[-2::-1]:
        p = p * u + np.float32(c)
    fc = jnp.where(d2 < np.float32(_RC * _RC), p, 0.0)
    logfc = jnp.where(fc > 0, jnp.log(fc), np.float32(-1e30))
    out_ref[0] = d2
    out_ref[1] = dist
    out_ref[2] = logfc


_scal_call = pl.pallas_call(
    _scal_body,
    grid=(_SROWS // _SB,),
    in_specs=[pl.BlockSpec((3, _SB, _SCOLS), lambda i: (0, i, 0))],
    out_specs=pl.BlockSpec((3, _SB, _SCOLS), lambda i: (0, i, 0)),
    out_shape=jax.ShapeDtypeStruct((3, _SROWS, _SCOLS), jnp.float32),
)


_PB = 2048


def _pair_body(a_ref, out_ref):
    a = a_ref[...]
    col = lax.broadcasted_iota(jnp.int32, (3, _NFP), 1)
    row = lax.broadcasted_iota(jnp.int32, (3, _NFP), 0)
    eta = 0.5 * jnp.exp2((col // len(_RS)).astype(jnp.float32))
    rs = 0.5 + (col % len(_RS)).astype(jnp.float32) * np.float32(4.0 / 7.0)
    cmat = jnp.where(row == 0, -eta, jnp.where(row == 1, 2.0 * eta * rs, 1.0))
    c0 = (-eta * rs * rs)[0:1]
    z = jnp.dot(a, cmat, preferred_element_type=jnp.float32,
                precision=lax.Precision.HIGHEST) + c0
    out_ref[...] = jnp.exp(z)


_pair_call = pl.pallas_call(
    _pair_body,
    grid=(_P_PAD // _PB,),
    in_specs=[pl.BlockSpec((_PB, 3), lambda i: (i, 0))],
    out_specs=pl.BlockSpec((_PB, _NFP), lambda i: (i, 0)),
    out_shape=jax.ShapeDtypeStruct((_P_PAD, _NFP), jnp.float32),
)


def _sc_body(sbf_hbm, idx_hbm, zeros_hbm, out_hbm, idx_v, buf, acc_sh):
    cid = lax.axis_index("c")
    sid = lax.axis_index("s")
    wid = cid * _NS + sid
    base = wid * _ROWS_W
    pltpu.sync_copy(zeros_hbm.at[pl.ds(sid * _AZ, _AZ)],
                    acc_sh.at[pl.ds(sid * _AZ, _AZ)])
    pltpu.sync_copy(idx_hbm.at[wid], idx_v)
    plsc.subcore_barrier()

    def g_body(g, carry):
        pltpu.sync_copy(sbf_hbm.at[pl.ds(base + g * _G, _G)], buf)
        for j in range(_NSUB):
            pltpu.sync_copy(buf.at[pl.ds(j * _SUB, _SUB)],
                            acc_sh.at[idx_v.at[g * _NSUB + j]],
                            add=True)
        return carry

    lax.fori_loop(0, _NG, g_body, 0)
    plsc.subcore_barrier()
    pltpu.sync_copy(acc_sh.at[pl.ds(sid * _AZ, _AZ)],
                    out_hbm.at[cid, pl.ds(sid * _AZ, _AZ)])


_sc_call_cache = []


def _get_sc_call():
    if not _sc_call_cache:
        _sc_call_cache.append(pl.kernel(
            _sc_body,
            out_type=jax.ShapeDtypeStruct((_NC, _NATOMS_PAD, _NFP), jnp.float32),
            mesh=plsc.VectorSubcoreMesh(core_axis_name="c",
                                        subcore_axis_name="s",
                                        num_cores=_NC, num_subcores=_NS),
            scratch_types=[
                pltpu.VMEM((_NG * _NSUB, _SUB), jnp.int32),
                pltpu.VMEM((_G, _NFP), jnp.float32),
                pltpu.VMEM_SHARED((_NATOMS_PAD, _NFP), jnp.float32),
            ],
            compiler_params=pltpu.CompilerParams(use_tc_tiling_on_sc=False),
        ))
    return _sc_call_cache[0]


_AB = 1000


def _mlp_body(fp0_ref, fp1_ref, e_ref, w0, b0, w1, b1, w2, b2, wo, out_ref):
    fp = (fp0_ref[...] + fp1_ref[...]) * np.float32(0.2) - 1.0
    h = jnp.tanh(jnp.dot(fp, w0[...], preferred_element_type=jnp.float32) + b0[...])
    h = jnp.tanh(jnp.dot(h, w1[...], preferred_element_type=jnp.float32) + b1[...])
    h = jnp.tanh(jnp.dot(h, w2[...], preferred_element_type=jnp.float32) + b2[...])
    e1 = e_ref[...] >= 0.5
    col1 = lax.broadcasted_iota(jnp.int32, (_AB, 256), 1) >= 128
    h = jnp.where(col1 == e1, h, 0.0)
    out_ref[...] = jnp.dot(h, wo[...], preferred_element_type=jnp.float32)


_mlp_call = pl.pallas_call(
    _mlp_body,
    grid=(_NATOMS // _AB,),
    in_specs=[
        pl.BlockSpec((_AB, _NFP), lambda i: (i, 0)),
        pl.BlockSpec((_AB, _NFP), lambda i: (i, 0)),
        pl.BlockSpec((_AB, 1), lambda i: (i, 0)),
        pl.BlockSpec((_NFP, 256), lambda i: (0, 0)),
        pl.BlockSpec((1, 256), lambda i: (0, 0)),
        pl.BlockSpec((256, 256), lambda i: (0, 0)),
        pl.BlockSpec((1, 256), lambda i: (0, 0)),
        pl.BlockSpec((256, 256), lambda i: (0, 0)),
        pl.BlockSpec((1, 256), lambda i: (0, 0)),
        pl.BlockSpec((256, 1), lambda i: (0, 0)),
    ],
    out_specs=pl.BlockSpec((_AB, 1), lambda i: (i, 0)),
    out_shape=jax.ShapeDtypeStruct((_NATOMS, 1), jnp.float32),
)


def kernel(diff, ind_2, elems,
           W0_0, b0_0, W1_0, b1_0, W2_0, b2_0, Wo_0,
           W0_1, b0_1, W1_1, b1_1, W2_1, b2_1, Wo_1):
    npad = _P_PAD - _NPAIRS
    diff_pad = jnp.concatenate(
        [diff, jnp.full((npad, 3), 100.0, jnp.float32)], axis=0)
    dxyz = diff_pad.T.reshape(3, _SROWS, _SCOLS)
    scal = _scal_call(dxyz)
    a2 = scal.reshape(3, _P_PAD).T
    sbf = _pair_call(a2)

    pad_idx = _NATOMS + jnp.arange(npad, dtype=jnp.int32) % (_NATOMS_PAD - _NATOMS)
    idx = jnp.concatenate([ind_2[:, 0], pad_idx]).reshape(_NW, _NG * _NSUB, _SUB)
    zeros = jnp.zeros((_NATOMS_PAD, _NFP), jnp.float32)
    fp_parts = _get_sc_call()(sbf, idx, zeros)

    w0 = jnp.concatenate([W0_0, W0_1], axis=1)
    b0 = jnp.concatenate([b0_0, b0_1]).reshape(1, 256)
    z = jnp.zeros((128, 128), jnp.float32)
    w1 = jnp.concatenate([jnp.concatenate([W1_0, z], 1),
                          jnp.concatenate([z, W1_1], 1)], 0)
    b1 = jnp.concatenate([b1_0, b1_1]).reshape(1, 256)
    w2 = jnp.concatenate([jnp.concatenate([W2_0, z], 1),
                          jnp.concatenate([z, W2_1], 1)], 0)
    b2 = jnp.concatenate([b2_0, b2_1]).reshape(1, 256)
    wo = jnp.concatenate([Wo_0, Wo_1], axis=0)
    e = elems.astype(jnp.float32).reshape(_NATOMS, 1)

    return _mlp_call(fp_parts[0], fp_parts[1], e, w0, b0, w1, b1, w2, b2, wo)


# speedup vs baseline: 1.0229x; 1.0229x over previous
import math

import numpy as np
import jax
import jax.numpy as jnp
from jax import lax
from jax.experimental import pallas as pl
from jax.experimental.pallas import tpu as pltpu, tpu_sc as plsc

_RC = 5.0
_RS = np.linspace(0.5, 4.5, 8).astype(np.float32)
_ETAS = np.array([0.5, 1.0, 2.0], dtype=np.float32)
_NFP = 24
_NATOMS = 10000
_NPAIRS = 320000

_NC, _NS = 2, 16
_NW = _NC * _NS
_ROWS_W = 10240
_P_PAD = _NW * _ROWS_W
_G = 1024
_NG = _ROWS_W // _G
_SUB = 128
_NSUB = _G // _SUB
_NATOMS_PAD = 10240
_AZ = _NATOMS_PAD // _NS


_SROWS = 640
_SCOLS = 512
_SB = 128
_FC_COEFFS = [1.0] + [0.5 * (-1.0) ** k / float(math.factorial(2 * k))
               for k in range(1, 9)]


def _scal_body(dxyz_ref, out_ref):
    dx = dxyz_ref[0]
    dy = dxyz_ref[1]
    dz = dxyz_ref[2]
    d2 = dx * dx + dy * dy + dz * dz
    dist = jnp.sqrt(d2 + 1e-12)
    u = d2 * np.float32((np.pi / _RC) ** 2)
    p = jnp.full_like(u, np.float32(_FC_COEFFS[-1]))
    for c in _FC_COEFFS[-2::-1]:
        p = p * u + np.float32(c)
    fc = jnp.where(d2 < np.float32(_RC * _RC), p, 0.0)
    logfc = jnp.where(fc > 0, jnp.log(fc), np.float32(-1e30))
    out_ref[0] = d2
    out_ref[1] = dist
    out_ref[2] = logfc


_scal_call = pl.pallas_call(
    _scal_body,
    grid=(_SROWS // _SB,),
    in_specs=[pl.BlockSpec((3, _SB, _SCOLS), lambda i: (0, i, 0))],
    out_specs=pl.BlockSpec((3, _SB, _SCOLS), lambda i: (0, i, 0)),
    out_shape=jax.ShapeDtypeStruct((3, _SROWS, _SCOLS), jnp.float32),
)


_PB = 2048


def _pair_body(a_ref, out_ref):
    a = a_ref[...]
    col = lax.broadcasted_iota(jnp.int32, (3, _NFP), 1)
    row = lax.broadcasted_iota(jnp.int32, (3, _NFP), 0)
    eta = 0.5 * jnp.exp2((col // len(_RS)).astype(jnp.float32))
    rs = 0.5 + (col % len(_RS)).astype(jnp.float32) * np.float32(4.0 / 7.0)
    cmat = jnp.where(row == 0, -eta, jnp.where(row == 1, 2.0 * eta * rs, 1.0))
    c0 = (-eta * rs * rs)[0:1]
    z = jnp.dot(a, cmat, preferred_element_type=jnp.float32,
                precision=lax.Precision.HIGHEST) + c0
    out_ref[...] = jnp.exp(z)


_pair_call = pl.pallas_call(
    _pair_body,
    grid=(_P_PAD // _PB,),
    in_specs=[pl.BlockSpec((_PB, 3), lambda i: (i, 0))],
    out_specs=pl.BlockSpec((_PB, _NFP), lambda i: (i, 0)),
    out_shape=jax.ShapeDtypeStruct((_P_PAD, _NFP), jnp.float32),
)


def _sc_body(sbf_hbm, idx_hbm, zeros_hbm, out_hbm, idx_v, bufa, bufb, acc_sh,
             gsem_a, gsem_b, ssem):
    cid = lax.axis_index("c")
    sid = lax.axis_index("s")
    wid = cid * _NS + sid
    base = wid * _ROWS_W
    pltpu.sync_copy(zeros_hbm.at[pl.ds(sid * _AZ, _AZ)],
                    acc_sh.at[pl.ds(sid * _AZ, _AZ)])
    pltpu.sync_copy(idx_hbm.at[wid], idx_v)
    plsc.subcore_barrier()

    def gather(g, buf, sem):
        pltpu.async_copy(sbf_hbm.at[pl.ds(base + g * _G, _G)], buf, sem)

    def wait(buf, sem):
        pltpu.make_async_copy(sbf_hbm.at[pl.ds(0, _G)], buf, sem).wait()

    def scatter_chunk(g, buf):
        for j in range(_NSUB):
            pltpu.async_copy(buf.at[pl.ds(j * _SUB, _SUB)],
                             acc_sh.at[idx_v.at[g * _NSUB + j]],
                             ssem, add=True)
        wait(buf, ssem)

    gather(0, bufa, gsem_a)
    gather(1, bufb, gsem_b)

    def g_body(i, carry):
        g = 2 * i
        wait(bufa, gsem_a)
        scatter_chunk(g, bufa)

        @pl.when(g + 2 < _NG)
        def _():
            gather(g + 2, bufa, gsem_a)

        wait(bufb, gsem_b)
        scatter_chunk(g + 1, bufb)

        @pl.when(g + 3 < _NG)
        def _():
            gather(g + 3, bufb, gsem_b)

        return carry

    lax.fori_loop(0, _NG // 2, g_body, 0)
    plsc.subcore_barrier()
    pltpu.sync_copy(acc_sh.at[pl.ds(sid * _AZ, _AZ)],
                    out_hbm.at[cid, pl.ds(sid * _AZ, _AZ)])


_sc_call_cache = []


def _get_sc_call():
    if not _sc_call_cache:
        _sc_call_cache.append(pl.kernel(
            _sc_body,
            out_type=jax.ShapeDtypeStruct((_NC, _NATOMS_PAD, _NFP), jnp.float32),
            mesh=plsc.VectorSubcoreMesh(core_axis_name="c",
                                        subcore_axis_name="s",
                                        num_cores=_NC, num_subcores=_NS),
            scratch_types=[
                pltpu.VMEM((_NG * _NSUB, _SUB), jnp.int32),
                pltpu.VMEM((_G, _NFP), jnp.float32),
                pltpu.VMEM((_G, _NFP), jnp.float32),
                pltpu.VMEM_SHARED((_NATOMS_PAD, _NFP), jnp.float32),
                pltpu.SemaphoreType.DMA,
                pltpu.SemaphoreType.DMA,
                pltpu.SemaphoreType.DMA,
            ],
            compiler_params=pltpu.CompilerParams(use_tc_tiling_on_sc=False),
        ))
    return _sc_call_cache[0]


_AB = 1000


def _mlp_body(fp0_ref, fp1_ref, e_ref, w0, b0, w1, b1, w2, b2, wo, out_ref):
    fp = (fp0_ref[...] + fp1_ref[...]) * np.float32(0.2) - 1.0
    h = jnp.tanh(jnp.dot(fp, w0[...], preferred_element_type=jnp.float32) + b0[...])
    h = jnp.tanh(jnp.dot(h, w1[...], preferred_element_type=jnp.float32) + b1[...])
    h = jnp.tanh(jnp.dot(h, w2[...], preferred_element_type=jnp.float32) + b2[...])
    e1 = e_ref[...] >= 0.5
    col1 = lax.broadcasted_iota(jnp.int32, (_AB, 256), 1) >= 128
    h = jnp.where(col1 == e1, h, 0.0)
    out_ref[...] = jnp.dot(h, wo[...], preferred_element_type=jnp.float32)


_mlp_call = pl.pallas_call(
    _mlp_body,
    grid=(_NATOMS // _AB,),
    in_specs=[
        pl.BlockSpec((_AB, _NFP), lambda i: (i, 0)),
        pl.BlockSpec((_AB, _NFP), lambda i: (i, 0)),
        pl.BlockSpec((_AB, 1), lambda i: (i, 0)),
        pl.BlockSpec((_NFP, 256), lambda i: (0, 0)),
        pl.BlockSpec((1, 256), lambda i: (0, 0)),
        pl.BlockSpec((256, 256), lambda i: (0, 0)),
        pl.BlockSpec((1, 256), lambda i: (0, 0)),
        pl.BlockSpec((256, 256), lambda i: (0, 0)),
        pl.BlockSpec((1, 256), lambda i: (0, 0)),
        pl.BlockSpec((256, 1), lambda i: (0, 0)),
    ],
    out_specs=pl.BlockSpec((_AB, 1), lambda i: (i, 0)),
    out_shape=jax.ShapeDtypeStruct((_NATOMS, 1), jnp.float32),
)


def kernel(diff, ind_2, elems,
           W0_0, b0_0, W1_0, b1_0, W2_0, b2_0, Wo_0,
           W0_1, b0_1, W1_1, b1_1, W2_1, b2_1, Wo_1):
    npad = _P_PAD - _NPAIRS
    diff_pad = jnp.concatenate(
        [diff, jnp.full((npad, 3), 100.0, jnp.float32)], axis=0)
    dxyz = diff_pad.T.reshape(3, _SROWS, _SCOLS)
    scal = _scal_call(dxyz)
    a2 = scal.reshape(3, _P_PAD).T
    sbf = _pair_call(a2)

    pad_idx = _NATOMS + jnp.arange(npad, dtype=jnp.int32) % (_NATOMS_PAD - _NATOMS)
    idx = jnp.concatenate([ind_2[:, 0], pad_idx]).reshape(_NW, _NG * _NSUB, _SUB)
    zeros = jnp.zeros((_NATOMS_PAD, _NFP), jnp.float32)
    fp_parts = _get_sc_call()(sbf, idx, zeros)

    w0 = jnp.concatenate([W0_0, W0_1], axis=1)
    b0 = jnp.concatenate([b0_0, b0_1]).reshape(1, 256)
    z = jnp.zeros((128, 128), jnp.float32)
    w1 = jnp.concatenate([jnp.concatenate([W1_0, z], 1),
                          jnp.concatenate([z, W1_1], 1)], 0)
    b1 = jnp.concatenate([b1_0, b1_1]).reshape(1, 256)
    w2 = jnp.concatenate([jnp.concatenate([W2_0, z], 1),
                          jnp.concatenate([z, W2_1], 1)], 0)
    b2 = jnp.concatenate([b2_0, b2_1]).reshape(1, 256)
    wo = jnp.concatenate([Wo_0, Wo_1], axis=0)
    e = elems.astype(jnp.float32).reshape(_NATOMS, 1)

    return _mlp_call(fp_parts[0], fp_parts[1], e, w0, b0, w1, b1, w2, b2, wo)


# speedup vs baseline: 1.3104x; 1.2810x over previous
import math

import numpy as np
import jax
import jax.numpy as jnp
from jax import lax
from jax.experimental import pallas as pl
from jax.experimental.pallas import tpu as pltpu, tpu_sc as plsc

_RC = 5.0
_RS = np.linspace(0.5, 4.5, 8).astype(np.float32)
_ETAS = np.array([0.5, 1.0, 2.0], dtype=np.float32)
_NFP = 24
_NATOMS = 10000
_NPAIRS = 320000

_NC, _NS = 2, 16
_NW = _NC * _NS
_ROWS_W = 10240
_P_PAD = _NW * _ROWS_W
_G = 1024
_NG = _ROWS_W // _G
_SUB = 128
_NSUB = _G // _SUB
_NATOMS_PAD = 10240
_AZ = _NATOMS_PAD // _NS


_SROWS = 640
_SCOLS = 512
_SB = 128
_FC_COEFFS = [1.0] + [0.5 * (-1.0) ** k / float(math.factorial(2 * k))
               for k in range(1, 9)]


def _scal_body(dxyz_ref, out_ref):
    dx = dxyz_ref[0]
    dy = dxyz_ref[1]
    dz = dxyz_ref[2]
    d2 = dx * dx + dy * dy + dz * dz
    dist = jnp.sqrt(d2 + 1e-12)
    u = d2 * np.float32((np.pi / _RC) ** 2)
    p = jnp.full_like(u, np.float32(_FC_COEFFS[-1]))
    for c in _FC_COEFFS[-2::-1]:
        p = p * u + np.float32(c)
    fc = jnp.where(d2 < np.float32(_RC * _RC), p, 0.0)
    logfc = jnp.where(fc > 0, jnp.log(fc), np.float32(-1e30))
    out_ref[0] = d2
    out_ref[1] = dist
    out_ref[2] = logfc


_scal_call = pl.pallas_call(
    _scal_body,
    grid=(_SROWS // _SB,),
    in_specs=[pl.BlockSpec((3, _SB, _SCOLS), lambda i: (0, i, 0))],
    out_specs=pl.BlockSpec((3, _SB, _SCOLS), lambda i: (0, i, 0)),
    out_shape=jax.ShapeDtypeStruct((3, _SROWS, _SCOLS), jnp.float32),
)


_PB = 2048


def _pair_body(a_ref, out_ref):
    a_t = a_ref[...]
    col = lax.broadcasted_iota(jnp.int32, (3, _NFP), 1)
    row = lax.broadcasted_iota(jnp.int32, (3, _NFP), 0)
    eta = 0.5 * jnp.exp2((col // len(_RS)).astype(jnp.float32))
    rs = 0.5 + (col % len(_RS)).astype(jnp.float32) * np.float32(4.0 / 7.0)
    cmat = jnp.where(row == 0, -eta, jnp.where(row == 1, 2.0 * eta * rs, 1.0))
    c0 = (-eta * rs * rs)[0:1]
    z = lax.dot_general(a_t, cmat, (((0,), (0,)), ((), ())),
                        preferred_element_type=jnp.float32,
                        precision=lax.Precision.HIGHEST) + c0
    out_ref[...] = jnp.exp(z)


_pair_call = pl.pallas_call(
    _pair_body,
    grid=(_P_PAD // _PB,),
    in_specs=[pl.BlockSpec((3, _PB), lambda i: (0, i))],
    out_specs=pl.BlockSpec((_PB, _NFP), lambda i: (i, 0)),
    out_shape=jax.ShapeDtypeStruct((_P_PAD, _NFP), jnp.float32),
)


def _sc_body(sbf_hbm, idx_hbm, zeros_hbm, out_hbm, idx_v, bufa, bufb, acc_sh,
             gsem_a, gsem_b, ssem):
    cid = lax.axis_index("c")
    sid = lax.axis_index("s")
    wid = cid * _NS + sid
    base = wid * _ROWS_W
    pltpu.sync_copy(zeros_hbm.at[pl.ds(sid * _AZ, _AZ)],
                    acc_sh.at[pl.ds(sid * _AZ, _AZ)])
    pltpu.sync_copy(idx_hbm.at[wid], idx_v)
    plsc.subcore_barrier()

    def gather(g, buf, sem):
        pltpu.async_copy(sbf_hbm.at[pl.ds(base + g * _G, _G)], buf, sem)

    def wait(buf, sem):
        pltpu.make_async_copy(sbf_hbm.at[pl.ds(0, _G)], buf, sem).wait()

    def scatter_chunk(g, buf):
        for j in range(_NSUB):
            pltpu.async_copy(buf.at[pl.ds(j * _SUB, _SUB)],
                             acc_sh.at[idx_v.at[g * _NSUB + j]],
                             ssem, add=True)
        wait(buf, ssem)

    gather(0, bufa, gsem_a)
    gather(1, bufb, gsem_b)

    def g_body(i, carry):
        g = 2 * i
        wait(bufa, gsem_a)
        scatter_chunk(g, bufa)

        @pl.when(g + 2 < _NG)
        def _():
            gather(g + 2, bufa, gsem_a)

        wait(bufb, gsem_b)
        scatter_chunk(g + 1, bufb)

        @pl.when(g + 3 < _NG)
        def _():
            gather(g + 3, bufb, gsem_b)

        return carry

    lax.fori_loop(0, _NG // 2, g_body, 0)
    plsc.subcore_barrier()
    pltpu.sync_copy(acc_sh.at[pl.ds(sid * _AZ, _AZ)],
                    out_hbm.at[cid, pl.ds(sid * _AZ, _AZ)])


_sc_call_cache = []


def _get_sc_call():
    if not _sc_call_cache:
        _sc_call_cache.append(pl.kernel(
            _sc_body,
            out_type=jax.ShapeDtypeStruct((_NC, _NATOMS_PAD, _NFP), jnp.float32),
            mesh=plsc.VectorSubcoreMesh(core_axis_name="c",
                                        subcore_axis_name="s",
                                        num_cores=_NC, num_subcores=_NS),
            scratch_types=[
                pltpu.VMEM((_NG * _NSUB, _SUB), jnp.int32),
                pltpu.VMEM((_G, _NFP), jnp.float32),
                pltpu.VMEM((_G, _NFP), jnp.float32),
                pltpu.VMEM_SHARED((_NATOMS_PAD, _NFP), jnp.float32),
                pltpu.SemaphoreType.DMA,
                pltpu.SemaphoreType.DMA,
                pltpu.SemaphoreType.DMA,
            ],
            compiler_params=pltpu.CompilerParams(use_tc_tiling_on_sc=False),
        ))
    return _sc_call_cache[0]


_AB = 1000


def _mlp_body(fp0_ref, fp1_ref, e_ref, w0, b0, w1, b1, w2, b2, wo, out_ref):
    fp = (fp0_ref[...] + fp1_ref[...]) * np.float32(0.2) - 1.0
    h = jnp.tanh(jnp.dot(fp, w0[...], preferred_element_type=jnp.float32) + b0[...])
    h = jnp.tanh(jnp.dot(h, w1[...], preferred_element_type=jnp.float32) + b1[...])
    h = jnp.tanh(jnp.dot(h, w2[...], preferred_element_type=jnp.float32) + b2[...])
    e1 = e_ref[...] >= 0.5
    col1 = lax.broadcasted_iota(jnp.int32, (_AB, 256), 1) >= 128
    h = jnp.where(col1 == e1, h, 0.0)
    out_ref[...] = jnp.dot(h, wo[...], preferred_element_type=jnp.float32)


_mlp_call = pl.pallas_call(
    _mlp_body,
    grid=(_NATOMS // _AB,),
    in_specs=[
        pl.BlockSpec((_AB, _NFP), lambda i: (i, 0)),
        pl.BlockSpec((_AB, _NFP), lambda i: (i, 0)),
        pl.BlockSpec((_AB, 1), lambda i: (i, 0)),
        pl.BlockSpec((_NFP, 256), lambda i: (0, 0)),
        pl.BlockSpec((1, 256), lambda i: (0, 0)),
        pl.BlockSpec((256, 256), lambda i: (0, 0)),
        pl.BlockSpec((1, 256), lambda i: (0, 0)),
        pl.BlockSpec((256, 256), lambda i: (0, 0)),
        pl.BlockSpec((1, 256), lambda i: (0, 0)),
        pl.BlockSpec((256, 1), lambda i: (0, 0)),
    ],
    out_specs=pl.BlockSpec((_AB, 1), lambda i: (i, 0)),
    out_shape=jax.ShapeDtypeStruct((_NATOMS, 1), jnp.float32),
)


def kernel(diff, ind_2, elems,
           W0_0, b0_0, W1_0, b1_0, W2_0, b2_0, Wo_0,
           W0_1, b0_1, W1_1, b1_1, W2_1, b2_1, Wo_1):
    npad = _P_PAD - _NPAIRS
    diff_pad = jnp.concatenate(
        [diff, jnp.full((npad, 3), 100.0, jnp.float32)], axis=0)
    dxyz = diff_pad.T.reshape(3, _SROWS, _SCOLS)
    scal = _scal_call(dxyz)
    a2t = scal.reshape(3, _P_PAD)
    sbf = _pair_call(a2t)

    pad_idx = _NATOMS + jnp.arange(npad, dtype=jnp.int32) % (_NATOMS_PAD - _NATOMS)
    idx = jnp.concatenate([ind_2[:, 0], pad_idx]).reshape(_NW, _NG * _NSUB, _SUB)
    zeros = jnp.zeros((_NATOMS_PAD, _NFP), jnp.float32)
    fp_parts = _get_sc_call()(sbf, idx, zeros)

    w0 = jnp.concatenate([W0_0, W0_1], axis=1)
    b0 = jnp.concatenate([b0_0, b0_1]).reshape(1, 256)
    z = jnp.zeros((128, 128), jnp.float32)
    w1 = jnp.concatenate([jnp.concatenate([W1_0, z], 1),
                          jnp.concatenate([z, W1_1], 1)], 0)
    b1 = jnp.concatenate([b1_0, b1_1]).reshape(1, 256)
    w2 = jnp.concatenate([jnp.concatenate([W2_0, z], 1),
                          jnp.concatenate([z, W2_1], 1)], 0)
    b2 = jnp.concatenate([b2_0, b2_1]).reshape(1, 256)
    wo = jnp.concatenate([Wo_0, Wo_1], axis=0)
    e = elems.astype(jnp.float32).reshape(_NATOMS, 1)

    return _mlp_call(fp_parts[0], fp_parts[1], e, w0, b0, w1, b1, w2, b2, wo)


# speedup vs baseline: 2.4547x; 1.8732x over previous
import math

import numpy as np
import jax
import jax.numpy as jnp
from jax import lax
from jax.experimental import pallas as pl
from jax.experimental.pallas import tpu as pltpu, tpu_sc as plsc

_RC = 5.0
_RS = np.linspace(0.5, 4.5, 8).astype(np.float32)
_ETAS = np.array([0.5, 1.0, 2.0], dtype=np.float32)
_NFP = 24
_NFPP = 32
_PACK = 4
_NATOMS = 10000
_NPAIRS = 320000

_NC, _NS = 2, 16
_NW = _NC * _NS
_ROWS_W = 10240
_P_PAD = _NW * _ROWS_W
_QROWS = _P_PAD // _PACK
_G = 1024
_NG = _ROWS_W // _G
_SUB = 128
_NSUB = _G // _SUB
_NATOMS_PAD = 10240
_AZ = _NATOMS_PAD // _NS


_SROWS = 640
_SCOLS = 512
_SB = 128
_FC_COEFFS = [1.0] + [0.5 * (-1.0) ** k / float(math.factorial(2 * k))
               for k in range(1, 9)]


def _scal_body(dxyz_ref, out_ref):
    dx = dxyz_ref[0]
    dy = dxyz_ref[1]
    dz = dxyz_ref[2]
    d2 = dx * dx + dy * dy + dz * dz
    dist = jnp.sqrt(d2 + 1e-12)
    u = d2 * np.float32((np.pi / _RC) ** 2)
    p = jnp.full_like(u, np.float32(_FC_COEFFS[-1]))
    for c in _FC_COEFFS[-2::-1]:
        p = p * u + np.float32(c)
    fc = jnp.where(d2 < np.float32(_RC * _RC), p, 0.0)
    logfc = jnp.where(fc > 0, jnp.log(fc), np.float32(-1e30))
    out_ref[0] = d2
    out_ref[1] = dist
    out_ref[2] = logfc


_scal_call = pl.pallas_call(
    _scal_body,
    grid=(_SROWS // _SB,),
    in_specs=[pl.BlockSpec((3, _SB, _SCOLS), lambda i: (0, i, 0))],
    out_specs=pl.BlockSpec((3, _SB, _SCOLS), lambda i: (0, i, 0)),
    out_shape=jax.ShapeDtypeStruct((3, _SROWS, _SCOLS), jnp.float32),
)


_QB = 2048


def _pair_body(a_ref, out_ref):
    a = a_ref[...]
    r = lax.broadcasted_iota(jnp.int32, (12, 128), 0)
    cc = lax.broadcasted_iota(jnp.int32, (12, 128), 1)
    i_r, j_r = r // 3, r % 3
    s_c, k_c = cc // _NFPP, cc % _NFPP
    eta = 0.5 * jnp.exp2((k_c // len(_RS)).astype(jnp.float32))
    rs = 0.5 + (k_c % len(_RS)).astype(jnp.float32) * np.float32(4.0 / 7.0)
    coeff = jnp.where(j_r == 0, -eta,
                      jnp.where(j_r == 1, 2.0 * eta * rs, 1.0))
    cmat = jnp.where((i_r == s_c) & (k_c < _NFP), coeff, 0.0)
    kc0 = lax.broadcasted_iota(jnp.int32, (1, 128), 1) % _NFPP
    eta0 = 0.5 * jnp.exp2((kc0 // len(_RS)).astype(jnp.float32))
    rs0 = 0.5 + (kc0 % len(_RS)).astype(jnp.float32) * np.float32(4.0 / 7.0)
    c0 = jnp.where(kc0 < _NFP, -eta0 * rs0 * rs0, np.float32(-1e30))
    z = lax.dot_general(a, cmat, (((0,), (0,)), ((), ())),
                        preferred_element_type=jnp.float32,
                        precision=lax.Precision.HIGHEST) + c0
    out_ref[...] = jnp.exp(z)


_pair_call = pl.pallas_call(
    _pair_body,
    grid=(_QROWS // _QB,),
    in_specs=[pl.BlockSpec((12, _QB), lambda i: (0, i))],
    out_specs=pl.BlockSpec((_QB, 128), lambda i: (i, 0)),
    out_shape=jax.ShapeDtypeStruct((_QROWS, 128), jnp.float32),
)


def _sc_body(sbf_hbm, idx_hbm, zeros_hbm, out_hbm, idx_v, bufa, bufb, acc_sh,
             gsem_a, gsem_b, ssem):
    cid = lax.axis_index("c")
    sid = lax.axis_index("s")
    wid = cid * _NS + sid
    base = wid * _ROWS_W
    pltpu.sync_copy(zeros_hbm.at[pl.ds(sid * _AZ, _AZ)],
                    acc_sh.at[pl.ds(sid * _AZ, _AZ)])
    pltpu.sync_copy(idx_hbm.at[wid], idx_v)
    plsc.subcore_barrier()

    def gather(g, buf, sem):
        pltpu.async_copy(sbf_hbm.at[pl.ds(base + g * _G, _G)], buf, sem)

    def wait(buf, sem):
        pltpu.make_async_copy(sbf_hbm.at[pl.ds(0, _G)], buf, sem).wait()

    def scatter_chunk(g, buf):
        for j in range(_NSUB):
            pltpu.async_copy(buf.at[pl.ds(j * _SUB, _SUB)],
                             acc_sh.at[idx_v.at[g * _NSUB + j]],
                             ssem, add=True)
        wait(buf, ssem)

    gather(0, bufa, gsem_a)
    gather(1, bufb, gsem_b)

    def g_body(i, carry):
        g = 2 * i
        wait(bufa, gsem_a)
        scatter_chunk(g, bufa)

        @pl.when(g + 2 < _NG)
        def _():
            gather(g + 2, bufa, gsem_a)

        wait(bufb, gsem_b)
        scatter_chunk(g + 1, bufb)

        @pl.when(g + 3 < _NG)
        def _():
            gather(g + 3, bufb, gsem_b)

        return carry

    lax.fori_loop(0, _NG // 2, g_body, 0)
    plsc.subcore_barrier()
    pltpu.sync_copy(acc_sh.at[pl.ds(sid * _AZ, _AZ)],
                    out_hbm.at[cid, pl.ds(sid * _AZ, _AZ)])


_sc_call_cache = []


def _get_sc_call():
    if not _sc_call_cache:
        _sc_call_cache.append(pl.kernel(
            _sc_body,
            out_type=jax.ShapeDtypeStruct((_NC, _NATOMS_PAD, _NFPP),
                                          jnp.float32),
            mesh=plsc.VectorSubcoreMesh(core_axis_name="c",
                                        subcore_axis_name="s",
                                        num_cores=_NC, num_subcores=_NS),
            scratch_types=[
                pltpu.VMEM((_NG * _NSUB, _SUB), jnp.int32),
                pltpu.VMEM((_G, _NFPP), jnp.float32),
                pltpu.VMEM((_G, _NFPP), jnp.float32),
                pltpu.VMEM_SHARED((_NATOMS_PAD, _NFPP), jnp.float32),
                pltpu.SemaphoreType.DMA,
                pltpu.SemaphoreType.DMA,
                pltpu.SemaphoreType.DMA,
            ],
            compiler_params=pltpu.CompilerParams(use_tc_tiling_on_sc=False),
        ))
    return _sc_call_cache[0]


_AB = 1000


def _mlp_body(fp0_ref, fp1_ref, e_ref, w0, b0, w1, b1, w2, b2, wo, out_ref):
    fp = (fp0_ref[...] + fp1_ref[...]) * np.float32(0.2) - 1.0
    h = jnp.tanh(jnp.dot(fp, w0[...], preferred_element_type=jnp.float32) + b0[...])
    h = jnp.tanh(jnp.dot(h, w1[...], preferred_element_type=jnp.float32) + b1[...])
    h = jnp.tanh(jnp.dot(h, w2[...], preferred_element_type=jnp.float32) + b2[...])
    e1 = e_ref[...] >= 0.5
    col1 = lax.broadcasted_iota(jnp.int32, (_AB, 256), 1) >= 128
    h = jnp.where(col1 == e1, h, 0.0)
    out_ref[...] = jnp.dot(h, wo[...], preferred_element_type=jnp.float32)


_mlp_call = pl.pallas_call(
    _mlp_body,
    grid=(_NATOMS // _AB,),
    in_specs=[
        pl.BlockSpec((_AB, _NFPP), lambda i: (i, 0)),
        pl.BlockSpec((_AB, _NFPP), lambda i: (i, 0)),
        pl.BlockSpec((_AB, 1), lambda i: (i, 0)),
        pl.BlockSpec((_NFPP, 256), lambda i: (0, 0)),
        pl.BlockSpec((1, 256), lambda i: (0, 0)),
        pl.BlockSpec((256, 256), lambda i: (0, 0)),
        pl.BlockSpec((1, 256), lambda i: (0, 0)),
        pl.BlockSpec((256, 256), lambda i: (0, 0)),
        pl.BlockSpec((1, 256), lambda i: (0, 0)),
        pl.BlockSpec((256, 1), lambda i: (0, 0)),
    ],
    out_specs=pl.BlockSpec((_AB, 1), lambda i: (i, 0)),
    out_shape=jax.ShapeDtypeStruct((_NATOMS, 1), jnp.float32),
)


def kernel(diff, ind_2, elems,
           W0_0, b0_0, W1_0, b1_0, W2_0, b2_0, Wo_0,
           W0_1, b0_1, W1_1, b1_1, W2_1, b2_1, Wo_1):
    npad = _P_PAD - _NPAIRS
    diff_pad = jnp.concatenate(
        [diff, jnp.full((npad, 3), 100.0, jnp.float32)], axis=0)
    dxyz = diff_pad.T.reshape(3, _SROWS, _SCOLS)
    scal = _scal_call(dxyz)
    a12 = scal.reshape(3, _PACK, _QROWS).transpose(1, 0, 2).reshape(12, _QROWS)
    sbf = _pair_call(a12).reshape(_P_PAD, _NFPP)

    pad_idx = _NATOMS + jnp.arange(npad, dtype=jnp.int32) % (_NATOMS_PAD - _NATOMS)
    idx = (jnp.concatenate([ind_2[:, 0], pad_idx])
           .reshape(_PACK, _QROWS).transpose(1, 0)
           .reshape(_NW, _NG * _NSUB, _SUB))
    zeros = jnp.zeros((_NATOMS_PAD, _NFPP), jnp.float32)
    fp_parts = _get_sc_call()(sbf, idx, zeros)

    w0 = jnp.concatenate(
        [jnp.concatenate([W0_0, W0_1], axis=1),
         jnp.zeros((_NFPP - _NFP, 256), jnp.float32)], axis=0)
    b0 = jnp.concatenate([b0_0, b0_1]).reshape(1, 256)
    z = jnp.zeros((128, 128), jnp.float32)
    w1 = jnp.concatenate([jnp.concatenate([W1_0, z], 1),
                          jnp.concatenate([z, W1_1], 1)], 0)
    b1 = jnp.concatenate([b1_0, b1_1]).reshape(1, 256)
    w2 = jnp.concatenate([jnp.concatenate([W2_0, z], 1),
                          jnp.concatenate([z, W2_1], 1)], 0)
    b2 = jnp.concatenate([b2_0, b2_1]).reshape(1, 256)
    wo = jnp.concatenate([Wo_0, Wo_1], axis=0)
    e = elems.astype(jnp.float32).reshape(_NATOMS, 1)

    return _mlp_call(fp_parts[0], fp_parts[1], e, w0, b0, w1, b1, w2, b2, wo)


# speedup vs baseline: 2.5159x; 1.0250x over previous
import math

import numpy as np
import jax
import jax.numpy as jnp
from jax import lax
from jax.experimental import pallas as pl
from jax.experimental.pallas import tpu as pltpu, tpu_sc as plsc

_RC = 5.0
_RS = np.linspace(0.5, 4.5, 8).astype(np.float32)
_ETAS = np.array([0.5, 1.0, 2.0], dtype=np.float32)
_NFP = 24
_NFPP = 32
_PACK = 4
_NATOMS = 10000
_NPAIRS = 320000

_NC, _NS = 2, 16
_NW = _NC * _NS
_ROWS_W = 10240
_P_PAD = _NW * _ROWS_W
_QROWS = _P_PAD // _PACK
_G = 1024
_NG = _ROWS_W // _G
_SUB = 128
_NSUB = _G // _SUB
_NATOMS_PAD = 10240
_AZ = _NATOMS_PAD // _NS


_SROWS = 640
_SCOLS = 512
_SB = 128
_FC_COEFFS = [1.0] + [0.5 * (-1.0) ** k / float(math.factorial(2 * k))
               for k in range(1, 9)]


def _scal_body(dxyz_ref, out_ref):
    dx = dxyz_ref[0]
    dy = dxyz_ref[1]
    dz = dxyz_ref[2]
    d2 = dx * dx + dy * dy + dz * dz
    dist = jnp.sqrt(d2 + 1e-12)
    u = d2 * np.float32((np.pi / _RC) ** 2)
    p = jnp.full_like(u, np.float32(_FC_COEFFS[-1]))
    for c in _FC_COEFFS[-2::-1]:
        p = p * u + np.float32(c)
    fc = jnp.where(d2 < np.float32(_RC * _RC), p, 0.0)
    logfc = jnp.where(fc > 0, jnp.log(fc), np.float32(-1e30))
    out_ref[0] = d2
    out_ref[1] = dist
    out_ref[2] = logfc


_scal_call = pl.pallas_call(
    _scal_body,
    grid=(_SROWS // _SB,),
    in_specs=[pl.BlockSpec((3, _SB, _SCOLS), lambda i: (0, i, 0))],
    out_specs=pl.BlockSpec((3, _SB, _SCOLS), lambda i: (0, i, 0)),
    out_shape=jax.ShapeDtypeStruct((3, _SROWS, _SCOLS), jnp.float32),
)


_QB = 4096


def _pair_body(a_ref, out_ref):
    a = a_ref[...]
    r = lax.broadcasted_iota(jnp.int32, (12, 128), 0)
    cc = lax.broadcasted_iota(jnp.int32, (12, 128), 1)
    i_r, j_r = r // 3, r % 3
    s_c, k_c = cc // _NFPP, cc % _NFPP
    eta = 0.5 * jnp.exp2((k_c // len(_RS)).astype(jnp.float32))
    rs = 0.5 + (k_c % len(_RS)).astype(jnp.float32) * np.float32(4.0 / 7.0)
    coeff = jnp.where(j_r == 0, -eta,
                      jnp.where(j_r == 1, 2.0 * eta * rs, 1.0))
    cmat = jnp.where((i_r == s_c) & (k_c < _NFP), coeff, 0.0)
    kc0 = lax.broadcasted_iota(jnp.int32, (1, 128), 1) % _NFPP
    eta0 = 0.5 * jnp.exp2((kc0 // len(_RS)).astype(jnp.float32))
    rs0 = 0.5 + (kc0 % len(_RS)).astype(jnp.float32) * np.float32(4.0 / 7.0)
    c0 = jnp.where(kc0 < _NFP, -eta0 * rs0 * rs0, np.float32(-1e30))
    z = lax.dot_general(a, cmat, (((0,), (0,)), ((), ())),
                        preferred_element_type=jnp.float32,
                        precision=lax.Precision.HIGHEST) + c0
    out_ref[...] = jnp.exp(z)


_pair_call = pl.pallas_call(
    _pair_body,
    grid=(_QROWS // _QB,),
    in_specs=[pl.BlockSpec((12, _QB), lambda i: (0, i))],
    out_specs=pl.BlockSpec((_QB, 128), lambda i: (i, 0)),
    out_shape=jax.ShapeDtypeStruct((_QROWS, 128), jnp.float32),
)


def _sc_body(sbf_hbm, idx_hbm, zeros_hbm, out_hbm, idx_v, bufa, bufb, acc_sh,
             gsem_a, gsem_b, ssem):
    cid = lax.axis_index("c")
    sid = lax.axis_index("s")
    wid = cid * _NS + sid
    base = wid * _ROWS_W
    pltpu.sync_copy(zeros_hbm.at[pl.ds(sid * _AZ, _AZ)],
                    acc_sh.at[pl.ds(sid * _AZ, _AZ)])
    pltpu.sync_copy(idx_hbm.at[wid], idx_v)
    plsc.subcore_barrier()

    def gather(g, buf, sem):
        pltpu.async_copy(sbf_hbm.at[pl.ds(base + g * _G, _G)], buf, sem)

    def wait(buf, sem):
        pltpu.make_async_copy(sbf_hbm.at[pl.ds(0, _G)], buf, sem).wait()

    def scatter_chunk(g, buf):
        for j in range(_NSUB):
            pltpu.async_copy(buf.at[pl.ds(j * _SUB, _SUB)],
                             acc_sh.at[idx_v.at[g * _NSUB + j]],
                             ssem, add=True)
        wait(buf, ssem)

    gather(0, bufa, gsem_a)
    gather(1, bufb, gsem_b)

    def g_body(i, carry):
        g = 2 * i
        wait(bufa, gsem_a)
        scatter_chunk(g, bufa)

        @pl.when(g + 2 < _NG)
        def _():
            gather(g + 2, bufa, gsem_a)

        wait(bufb, gsem_b)
        scatter_chunk(g + 1, bufb)

        @pl.when(g + 3 < _NG)
        def _():
            gather(g + 3, bufb, gsem_b)

        return carry

    lax.fori_loop(0, _NG // 2, g_body, 0)
    plsc.subcore_barrier()
    pltpu.sync_copy(acc_sh.at[pl.ds(sid * _AZ, _AZ)],
                    out_hbm.at[cid, pl.ds(sid * _AZ, _AZ)])


_sc_call_cache = []


def _get_sc_call():
    if not _sc_call_cache:
        _sc_call_cache.append(pl.kernel(
            _sc_body,
            out_type=jax.ShapeDtypeStruct((_NC, _NATOMS_PAD, _NFPP),
                                          jnp.float32),
            mesh=plsc.VectorSubcoreMesh(core_axis_name="c",
                                        subcore_axis_name="s",
                                        num_cores=_NC, num_subcores=_NS),
            scratch_types=[
                pltpu.VMEM((_NG * _NSUB, _SUB), jnp.int32),
                pltpu.VMEM((_G, _NFPP), jnp.float32),
                pltpu.VMEM((_G, _NFPP), jnp.float32),
                pltpu.VMEM_SHARED((_NATOMS_PAD, _NFPP), jnp.float32),
                pltpu.SemaphoreType.DMA,
                pltpu.SemaphoreType.DMA,
                pltpu.SemaphoreType.DMA,
            ],
            compiler_params=pltpu.CompilerParams(use_tc_tiling_on_sc=False),
        ))
    return _sc_call_cache[0]


_AB = 1000


def _mlp_body(fp0_ref, fp1_ref, e_ref, w0, b0, w1, b1, w2, b2, wo, out_ref):
    fp = (fp0_ref[...] + fp1_ref[...]) * np.float32(0.2) - 1.0
    h = jnp.tanh(jnp.dot(fp, w0[...], preferred_element_type=jnp.float32) + b0[...])
    h = jnp.tanh(jnp.dot(h, w1[...], preferred_element_type=jnp.float32) + b1[...])
    h = jnp.tanh(jnp.dot(h, w2[...], preferred_element_type=jnp.float32) + b2[...])
    e1 = e_ref[...] >= 0.5
    col1 = lax.broadcasted_iota(jnp.int32, (_AB, 256), 1) >= 128
    h = jnp.where(col1 == e1, h, 0.0)
    out_ref[...] = jnp.dot(h, wo[...], preferred_element_type=jnp.float32)


_mlp_call = pl.pallas_call(
    _mlp_body,
    grid=(_NATOMS // _AB,),
    in_specs=[
        pl.BlockSpec((_AB, _NFPP), lambda i: (i, 0)),
        pl.BlockSpec((_AB, _NFPP), lambda i: (i, 0)),
        pl.BlockSpec((_AB, 1), lambda i: (i, 0)),
        pl.BlockSpec((_NFPP, 256), lambda i: (0, 0)),
        pl.BlockSpec((1, 256), lambda i: (0, 0)),
        pl.BlockSpec((256, 256), lambda i: (0, 0)),
        pl.BlockSpec((1, 256), lambda i: (0, 0)),
        pl.BlockSpec((256, 256), lambda i: (0, 0)),
        pl.BlockSpec((1, 256), lambda i: (0, 0)),
        pl.BlockSpec((256, 1), lambda i: (0, 0)),
    ],
    out_specs=pl.BlockSpec((_AB, 1), lambda i: (i, 0)),
    out_shape=jax.ShapeDtypeStruct((_NATOMS, 1), jnp.float32),
)


def kernel(diff, ind_2, elems,
           W0_0, b0_0, W1_0, b1_0, W2_0, b2_0, Wo_0,
           W0_1, b0_1, W1_1, b1_1, W2_1, b2_1, Wo_1):
    npad = _P_PAD - _NPAIRS
    diff_pad = jnp.concatenate(
        [diff, jnp.full((npad, 3), 100.0, jnp.float32)], axis=0)
    dxyz = diff_pad.T.reshape(3, _SROWS, _SCOLS)
    scal = _scal_call(dxyz)
    a12 = scal.reshape(3, _PACK, _QROWS).transpose(1, 0, 2).reshape(12, _QROWS)
    sbf = _pair_call(a12).reshape(_P_PAD, _NFPP)

    pad_idx = _NATOMS + jnp.arange(npad, dtype=jnp.int32) % (_NATOMS_PAD - _NATOMS)
    idx = (jnp.concatenate([ind_2[:, 0], pad_idx])
           .reshape(_PACK, _QROWS).transpose(1, 0)
           .reshape(_NW, _NG * _NSUB, _SUB))
    zeros = jnp.zeros((_NATOMS_PAD, _NFPP), jnp.float32)
    fp_parts = _get_sc_call()(sbf, idx, zeros)

    w0 = jnp.concatenate(
        [jnp.concatenate([W0_0, W0_1], axis=1),
         jnp.zeros((_NFPP - _NFP, 256), jnp.float32)], axis=0)
    b0 = jnp.concatenate([b0_0, b0_1]).reshape(1, 256)
    z = jnp.zeros((128, 128), jnp.float32)
    w1 = jnp.concatenate([jnp.concatenate([W1_0, z], 1),
                          jnp.concatenate([z, W1_1], 1)], 0)
    b1 = jnp.concatenate([b1_0, b1_1]).reshape(1, 256)
    w2 = jnp.concatenate([jnp.concatenate([W2_0, z], 1),
                          jnp.concatenate([z, W2_1], 1)], 0)
    b2 = jnp.concatenate([b2_0, b2_1]).reshape(1, 256)
    wo = jnp.concatenate([Wo_0, Wo_1], axis=0)
    e = elems.astype(jnp.float32).reshape(_NATOMS, 1)

    return _mlp_call(fp_parts[0], fp_parts[1], e, w0, b0, w1, b1, w2, b2, wo)


# speedup vs baseline: 2.5720x; 1.0223x over previous
import math

import numpy as np
import jax
import jax.numpy as jnp
from jax import lax
from jax.experimental import pallas as pl
from jax.experimental.pallas import tpu as pltpu, tpu_sc as plsc

_RC = 5.0
_RS = np.linspace(0.5, 4.5, 8).astype(np.float32)
_ETAS = np.array([0.5, 1.0, 2.0], dtype=np.float32)
_NFP = 24
_NFPP = 32
_PACK = 4
_NATOMS = 10000
_NPAIRS = 320000

_NC, _NS = 2, 16
_NW = _NC * _NS
_ROWS_W = 10240
_P_PAD = _NW * _ROWS_W
_QROWS = _P_PAD // _PACK
_G = 1024
_NG = _ROWS_W // _G
_SUB = 128
_NSUB = _G // _SUB
_NATOMS_PAD = 10240
_AZ = _NATOMS_PAD // _NS


_SROWS = 640
_SCOLS = 512
_SB = 128
_FC_COEFFS = [1.0] + [0.5 * (-1.0) ** k / float(math.factorial(2 * k))
               for k in range(1, 9)]


def _scal_body(dxyz_ref, out_ref):
    dx = dxyz_ref[0]
    dy = dxyz_ref[1]
    dz = dxyz_ref[2]
    d2 = dx * dx + dy * dy + dz * dz
    dist = jnp.sqrt(d2 + 1e-12)
    u = d2 * np.float32((np.pi / _RC) ** 2)
    p = jnp.full_like(u, np.float32(_FC_COEFFS[-1]))
    for c in _FC_COEFFS[-2::-1]:
        p = p * u + np.float32(c)
    fc = jnp.where(d2 < np.float32(_RC * _RC), p, 0.0)
    logfc = jnp.where(fc > 0, jnp.log(fc), np.float32(-1e30))
    out_ref[0] = d2
    out_ref[1] = dist
    out_ref[2] = logfc


_scal_call = pl.pallas_call(
    _scal_body,
    grid=(_SROWS // _SB,),
    in_specs=[pl.BlockSpec((3, _SB, _SCOLS), lambda i: (0, i, 0))],
    out_specs=pl.BlockSpec((3, _SB, _SCOLS), lambda i: (0, i, 0)),
    out_shape=jax.ShapeDtypeStruct((3, _SROWS, _SCOLS), jnp.float32),
)


_QB = 4096


def _pair_body(a_ref, out_ref):
    a = a_ref[...]
    r = lax.broadcasted_iota(jnp.int32, (12, 128), 0)
    cc = lax.broadcasted_iota(jnp.int32, (12, 128), 1)
    i_r, j_r = r // 3, r % 3
    s_c, k_c = cc // _NFPP, cc % _NFPP
    eta = 0.5 * jnp.exp2((k_c // len(_RS)).astype(jnp.float32))
    rs = 0.5 + (k_c % len(_RS)).astype(jnp.float32) * np.float32(4.0 / 7.0)
    coeff = jnp.where(j_r == 0, -eta,
                      jnp.where(j_r == 1, 2.0 * eta * rs, 1.0))
    cmat = jnp.where((i_r == s_c) & (k_c < _NFP), coeff, 0.0)
    kc0 = lax.broadcasted_iota(jnp.int32, (1, 128), 1) % _NFPP
    eta0 = 0.5 * jnp.exp2((kc0 // len(_RS)).astype(jnp.float32))
    rs0 = 0.5 + (kc0 % len(_RS)).astype(jnp.float32) * np.float32(4.0 / 7.0)
    c0 = jnp.where(kc0 < _NFP, -eta0 * rs0 * rs0, np.float32(-1e30))
    z = lax.dot_general(a, cmat, (((0,), (0,)), ((), ())),
                        preferred_element_type=jnp.float32,
                        precision=lax.Precision.HIGHEST) + c0
    out_ref[...] = jnp.exp(z)


_pair_call = pl.pallas_call(
    _pair_body,
    grid=(_QROWS // _QB,),
    in_specs=[pl.BlockSpec((12, _QB), lambda i: (0, i))],
    out_specs=pl.BlockSpec((_QB, 128), lambda i: (i, 0)),
    out_shape=jax.ShapeDtypeStruct((_QROWS, 128), jnp.float32),
)


def _sc_body(sbf_hbm, idx_hbm, zeros_hbm, out_hbm, idx_v, bufa, bufb, acc_sh,
             gsem_a, gsem_b, ssem):
    cid = lax.axis_index("c")
    sid = lax.axis_index("s")
    wid = cid * _NS + sid
    base = wid * _ROWS_W
    pltpu.sync_copy(zeros_hbm.at[pl.ds(sid * _AZ, _AZ)],
                    acc_sh.at[pl.ds(sid * _AZ, _AZ)])
    pltpu.sync_copy(idx_hbm.at[wid], idx_v)
    plsc.subcore_barrier()

    def gather(g, buf, sem):
        pltpu.async_copy(sbf_hbm.at[pl.ds(base + g * _G, _G)], buf, sem)

    def wait(buf, sem):
        pltpu.make_async_copy(sbf_hbm.at[pl.ds(0, _G)], buf, sem).wait()

    def scatter_chunk(g, buf):
        for j in range(_NSUB):
            pltpu.async_copy(buf.at[pl.ds(j * _SUB, _SUB)],
                             acc_sh.at[idx_v.at[g * _NSUB + j]],
                             ssem, add=True)
        wait(buf, ssem)

    gather(0, bufa, gsem_a)
    gather(1, bufb, gsem_b)

    def g_body(i, carry):
        g = 2 * i
        wait(bufa, gsem_a)
        scatter_chunk(g, bufa)

        @pl.when(g + 2 < _NG)
        def _():
            gather(g + 2, bufa, gsem_a)

        wait(bufb, gsem_b)
        scatter_chunk(g + 1, bufb)

        @pl.when(g + 3 < _NG)
        def _():
            gather(g + 3, bufb, gsem_b)

        return carry

    lax.fori_loop(0, _NG // 2, g_body, 0)
    plsc.subcore_barrier()
    pltpu.sync_copy(acc_sh.at[pl.ds(sid * _AZ, _AZ)],
                    out_hbm.at[cid, pl.ds(sid * _AZ, _AZ)])


_sc_call_cache = []


def _get_sc_call():
    if not _sc_call_cache:
        _sc_call_cache.append(pl.kernel(
            _sc_body,
            out_type=jax.ShapeDtypeStruct((_NC, _NATOMS_PAD, _NFPP),
                                          jnp.float32),
            mesh=plsc.VectorSubcoreMesh(core_axis_name="c",
                                        subcore_axis_name="s",
                                        num_cores=_NC, num_subcores=_NS),
            scratch_types=[
                pltpu.VMEM((_NG * _NSUB, _SUB), jnp.int32),
                pltpu.VMEM((_G, _NFPP), jnp.float32),
                pltpu.VMEM((_G, _NFPP), jnp.float32),
                pltpu.VMEM_SHARED((_NATOMS_PAD, _NFPP), jnp.float32),
                pltpu.SemaphoreType.DMA,
                pltpu.SemaphoreType.DMA,
                pltpu.SemaphoreType.DMA,
            ],
            compiler_params=pltpu.CompilerParams(use_tc_tiling_on_sc=False),
        ))
    return _sc_call_cache[0]


_AB = 1000


def _mlp_body(fpp0_ref, fpp1_ref, e_ref,
              w00, w01, b00, b01, w10, w11, b10, b11,
              w20, w21, b20, b21, wo0, wo1, out_ref):
    def dot(x, w):
        return jnp.dot(x, w[...], preferred_element_type=jnp.float32)

    fp = (fpp0_ref[0] + fpp1_ref[0]) * np.float32(0.2) - 1.0
    fp24 = fp[:, :_NFP]
    ha = jnp.tanh(dot(fp24, w00) + b00[...])
    hb = jnp.tanh(dot(fp24, w01) + b01[...])
    ha = jnp.tanh(dot(ha, w10) + b10[...])
    hb = jnp.tanh(dot(hb, w11) + b11[...])
    ha = jnp.tanh(dot(ha, w20) + b20[...])
    hb = jnp.tanh(dot(hb, w21) + b21[...])
    y0 = dot(ha, wo0)
    y1 = dot(hb, wo1)
    out_ref[...] = jnp.where(e_ref[...] >= 1, y1, y0)


def _wspec(shape):
    return pl.BlockSpec(shape, lambda i: tuple(0 for _ in shape))


_mlp_call = pl.pallas_call(
    _mlp_body,
    grid=(_NATOMS // _AB,),
    in_specs=[
        pl.BlockSpec((1, _AB, _NFPP), lambda i: (0, i, 0)),
        pl.BlockSpec((1, _AB, _NFPP), lambda i: (1, i, 0)),
        pl.BlockSpec((_AB, 1), lambda i: (i, 0)),
        _wspec((_NFP, 128)), _wspec((_NFP, 128)),
        _wspec((1, 128)), _wspec((1, 128)),
        _wspec((128, 128)), _wspec((128, 128)),
        _wspec((1, 128)), _wspec((1, 128)),
        _wspec((128, 128)), _wspec((128, 128)),
        _wspec((1, 128)), _wspec((1, 128)),
        _wspec((128, 1)), _wspec((128, 1)),
    ],
    out_specs=pl.BlockSpec((_AB, 1), lambda i: (i, 0)),
    out_shape=jax.ShapeDtypeStruct((_NATOMS, 1), jnp.float32),
)


def kernel(diff, ind_2, elems,
           W0_0, b0_0, W1_0, b1_0, W2_0, b2_0, Wo_0,
           W0_1, b0_1, W1_1, b1_1, W2_1, b2_1, Wo_1):
    npad = _P_PAD - _NPAIRS
    diff_pad = jnp.concatenate(
        [diff, jnp.full((npad, 3), 100.0, jnp.float32)], axis=0)
    dxyz = diff_pad.T.reshape(3, _SROWS, _SCOLS)
    scal = _scal_call(dxyz)
    a12 = scal.reshape(3, _PACK, _QROWS).transpose(1, 0, 2).reshape(12, _QROWS)
    sbf = _pair_call(a12).reshape(_P_PAD, _NFPP)

    pad_idx = _NATOMS + jnp.arange(npad, dtype=jnp.int32) % (_NATOMS_PAD - _NATOMS)
    idx = (jnp.concatenate([ind_2[:, 0], pad_idx])
           .reshape(_PACK, _QROWS).transpose(1, 0)
           .reshape(_NW, _NG * _NSUB, _SUB))
    zeros = jnp.zeros((_NATOMS_PAD, _NFPP), jnp.float32)
    fp_parts = _get_sc_call()(sbf, idx, zeros)

    e = elems.reshape(_NATOMS, 1)
    return _mlp_call(
        fp_parts, fp_parts, e,
        W0_0, W0_1, b0_0.reshape(1, 128), b0_1.reshape(1, 128),
        W1_0, W1_1, b1_0.reshape(1, 128), b1_1.reshape(1, 128),
        W2_0, W2_1, b2_0.reshape(1, 128), b2_1.reshape(1, 128),
        Wo_0, Wo_1)
